# Initial kernel scaffold; baseline (speedup 1.0000x reference)
import jax
import jax.numpy as jnp
from jax.experimental import pallas as pl


def kernel(x, edge_index, sage_w_l, sage_b_l, sage_w_r, gat_w, gat_att_src, gat_att_dst, gat_bias):
    raise NotImplementedError("write your pallas kernel here")


# speedup vs baseline: 11.4154x; 11.4154x over previous
import functools

import jax
import jax.numpy as jnp
from jax import lax
from jax.experimental import pallas as pl
from jax.experimental.pallas import tpu as pltpu
from jax.experimental.pallas import tpu_sc as plsc

N = 10000
D = 128
NP = 10240
DUMMY = 10000
E = 320000
NC, NS, L = 2, 16, 16
NWORK = NC * NS

CHUNK = 128
EP1 = 323584
EPW1 = EP1 // NWORK
C1 = EPW1 // CHUNK
EP2 = 331776
EPW2 = EP2 // NWORK
C2 = EPW2 // CHUNK
RPT = NP // NS
W16 = 16

_mesh = plsc.VectorSubcoreMesh(
    core_axis_name="c", subcore_axis_name="s", num_cores=NC, num_subcores=NS
)


def _zero_rows(ref, nrows, ncol16):
    def body(i, carry):
        for q in range(ncol16):
            ref[i, pl.ds(q * L, L)] = jnp.zeros((L,), jnp.float32)
        return carry
    lax.fori_loop(0, nrows, body, 0)


def _sage_mm_body(xp_ref, wlt_ref, wrt_ref, bl_ref, y_ref, r_ref):
    xb = xp_ref[...]
    y_ref[...] = lax.dot_general(
        xb, wlt_ref[...], (((1,), (0,)), ((), ())),
        preferred_element_type=jnp.float32)
    r_ref[...] = lax.dot_general(
        xb, wrt_ref[...], (((1,), (0,)), ((), ())),
        preferred_element_type=jnp.float32) + bl_ref[...]


_sage_mm = pl.pallas_call(
    _sage_mm_body,
    out_shape=(
        jax.ShapeDtypeStruct((NP, D), jnp.float32),
        jax.ShapeDtypeStruct((NP, D), jnp.float32),
    ),
)


@functools.partial(
    pl.kernel,
    out_type=(
        jax.ShapeDtypeStruct((NC, NP, D), jnp.float32),
        jax.ShapeDtypeStruct((NC, NP, W16), jnp.float32),
    ),
    mesh=_mesh,
    scratch_types=[
        pltpu.VMEM_SHARED((NP, D), jnp.float32),
        pltpu.VMEM_SHARED((NP, W16), jnp.float32),
        pltpu.VMEM((1, CHUNK), jnp.int32),
        pltpu.VMEM((1, CHUNK), jnp.int32),
        pltpu.VMEM((CHUNK, D), jnp.float32),
        pltpu.VMEM((CHUNK, W16), jnp.float32),
        pltpu.VMEM((CHUNK, W16), jnp.float32),
        pltpu.SemaphoreType.DMA,
    ],
    compiler_params=pltpu.CompilerParams(use_tc_tiling_on_sc=False, needs_layout_passes=False),
)
def _sage_sc(y_hbm, src_hbm, dst_hbm, accp_hbm, cntp_hbm,
             acc_sh, cnt_sh, srcv, dstv, rows_v, zbuf16, ones16, sem):
    cid = lax.axis_index("c")
    sid = lax.axis_index("s")
    wid = cid * NS + sid

    _zero_rows(rows_v, CHUNK, D // L)
    _zero_rows(zbuf16, CHUNK, 1)

    def fill_ones(i, carry):
        ones16[i, pl.ds(0, L)] = jnp.ones((L,), jnp.float32)
        return carry
    lax.fori_loop(0, CHUNK, fill_ones, 0)

    for k in range(RPT // CHUNK):
        pltpu.sync_copy(rows_v, acc_sh.at[pl.ds(sid * RPT + k * CHUNK, CHUNK)])
        pltpu.sync_copy(zbuf16, cnt_sh.at[pl.ds(sid * RPT + k * CHUNK, CHUNK)])
    plsc.subcore_barrier()

    base = wid * EPW1

    def body(k, carry):
        off = base + k * CHUNK
        pltpu.sync_copy(src_hbm.at[pl.ds(off, CHUNK)], srcv.at[0])
        pltpu.sync_copy(dst_hbm.at[pl.ds(off, CHUNK)], dstv.at[0])
        pltpu.async_copy(y_hbm.at[srcv.at[0]], rows_v, sem).wait()
        pltpu.sync_copy(rows_v, acc_sh.at[dstv.at[0]], add=True)
        pltpu.sync_copy(ones16, cnt_sh.at[dstv.at[0]], add=True)
        return carry
    lax.fori_loop(0, C1, body, 0)

    plsc.subcore_barrier()
    pltpu.sync_copy(acc_sh.at[pl.ds(sid * RPT, RPT)],
                    accp_hbm.at[cid, pl.ds(sid * RPT, RPT)])
    pltpu.sync_copy(cnt_sh.at[pl.ds(sid * RPT, RPT)],
                    cntp_hbm.at[cid, pl.ds(sid * RPT, RPT)])


def _mid_body(accp_ref, cntp_ref, r_ref, gwt_ref, ats_ref, atd_ref,
              h2_ref, as_ref, ad_ref, cs_ref):
    acc = accp_ref[0] + accp_ref[1]
    cnt = (cntp_ref[0] + cntp_ref[1])[:, 0:1]
    mean = acc / jnp.maximum(cnt, 1.0)
    h = jnp.maximum(mean + r_ref[...], 0.0)
    h2 = lax.dot_general(
        h, gwt_ref[...], (((1,), (0,)), ((), ())),
        preferred_element_type=jnp.float32)
    h2_ref[...] = h2
    a_s = jnp.sum(h2 * ats_ref[...], axis=1, keepdims=True)
    a_d = jnp.sum(h2 * atd_ref[...], axis=1, keepdims=True)
    ridx = lax.broadcasted_iota(jnp.int32, (NP, 1), 0)
    valid = ridx < N
    a_s = jnp.where(valid, a_s, -3e38)
    a_d = jnp.where(valid, a_d, -3e38)
    as_ref[...] = a_s
    ad_ref[...] = a_d
    m = jnp.max(a_s) + jnp.max(a_d)
    cs_ref[...] = jnp.full((1, D), jnp.where(m > 0.0, m, 0.2 * m),
                           jnp.float32)


_mid_tc = pl.pallas_call(
    _mid_body,
    out_shape=(
        jax.ShapeDtypeStruct((NP, D), jnp.float32),
        jax.ShapeDtypeStruct((NP, 1), jnp.float32),
        jax.ShapeDtypeStruct((NP, 1), jnp.float32),
        jax.ShapeDtypeStruct((1, D), jnp.float32),
    ),
)


@functools.partial(
    pl.kernel,
    out_type=(
        jax.ShapeDtypeStruct((NC, NP, D), jnp.float32),
        jax.ShapeDtypeStruct((NC, NP, W16), jnp.float32),
    ),
    mesh=_mesh,
    scratch_types=[
        pltpu.VMEM_SHARED((NP, D), jnp.float32),
        pltpu.VMEM_SHARED((NP, W16), jnp.float32),
        pltpu.VMEM((1, CHUNK), jnp.int32),
        pltpu.VMEM((1, CHUNK), jnp.int32),
        pltpu.VMEM((CHUNK,), jnp.float32),
        pltpu.VMEM((CHUNK,), jnp.float32),
        pltpu.VMEM((CHUNK,), jnp.float32),
        pltpu.VMEM((CHUNK, W16), jnp.float32),
        pltpu.VMEM((CHUNK, D), jnp.float32),
        pltpu.VMEM((CHUNK, W16), jnp.float32),
        pltpu.VMEM((L,), jnp.float32),
        pltpu.SemaphoreType.DMA,
    ],
    compiler_params=pltpu.CompilerParams(use_tc_tiling_on_sc=False, needs_layout_passes=False),
)
def _gat_sc(h2_hbm, as_hbm, ad_hbm, cs_hbm, src_hbm, dst_hbm, accp_hbm,
            sp_hbm, acc_sh, ssum_sh, srcv, dstv, asg, adg, ev, ebuf,
            rows_v, zbuf16, csv, sem):
    cid = lax.axis_index("c")
    sid = lax.axis_index("s")
    wid = cid * NS + sid

    _zero_rows(rows_v, CHUNK, D // L)
    _zero_rows(zbuf16, CHUNK, 1)
    for k in range(RPT // CHUNK):
        pltpu.sync_copy(rows_v, acc_sh.at[pl.ds(sid * RPT + k * CHUNK, CHUNK)])
        pltpu.sync_copy(zbuf16, ssum_sh.at[pl.ds(sid * RPT + k * CHUNK, CHUNK)])
    plsc.subcore_barrier()

    pltpu.sync_copy(cs_hbm.at[pl.ds(0, L)], csv)
    cshift = csv[pl.ds(0, L)]

    base = wid * EPW2

    def body(k, carry):
        off = base + k * CHUNK
        pltpu.sync_copy(src_hbm.at[pl.ds(off, CHUNK)], srcv.at[0])
        pltpu.sync_copy(dst_hbm.at[pl.ds(off, CHUNK)], dstv.at[0])
        pltpu.async_copy(h2_hbm.at[srcv.at[0]], rows_v, sem).wait()
        pltpu.async_copy(as_hbm.at[srcv.at[0]], asg, sem).wait()
        pltpu.async_copy(ad_hbm.at[dstv.at[0]], adg, sem).wait()
        for q in range(CHUNK // L):
            z = asg[pl.ds(q * L, L)] + adg[pl.ds(q * L, L)]
            z = jnp.where(z > 0.0, z, 0.2 * z) - cshift
            ev[pl.ds(q * L, L)] = jnp.exp(z)

        def scale(i, carry2):
            b = plsc.load_gather(ev, [jnp.full((L,), i, jnp.int32)])
            ebuf[i, pl.ds(0, L)] = b
            for q in range(D // L):
                rows_v[i, pl.ds(q * L, L)] = rows_v[i, pl.ds(q * L, L)] * b
            return carry2
        lax.fori_loop(0, CHUNK, scale, 0)

        pltpu.sync_copy(rows_v, acc_sh.at[dstv.at[0]], add=True)
        pltpu.sync_copy(ebuf, ssum_sh.at[dstv.at[0]], add=True)
        return carry
    lax.fori_loop(0, C2, body, 0)

    plsc.subcore_barrier()
    pltpu.sync_copy(acc_sh.at[pl.ds(sid * RPT, RPT)],
                    accp_hbm.at[cid, pl.ds(sid * RPT, RPT)])
    pltpu.sync_copy(ssum_sh.at[pl.ds(sid * RPT, RPT)],
                    sp_hbm.at[cid, pl.ds(sid * RPT, RPT)])


def _fin_body(accp_ref, sp_ref, bias_ref, out_ref):
    o = accp_ref[0] + accp_ref[1]
    s = (sp_ref[0] + sp_ref[1])[:, 0:1]
    o = o / (s + 1e-16) + bias_ref[...]
    mx = jnp.max(o, axis=1, keepdims=True)
    lse = jnp.log(jnp.sum(jnp.exp(o - mx), axis=1, keepdims=True)) + mx
    out_ref[...] = (o - lse)[:N]


_fin_tc = pl.pallas_call(
    _fin_body,
    out_shape=jax.ShapeDtypeStruct((N, D), jnp.float32),
)


def kernel(x, edge_index, sage_w_l, sage_b_l, sage_w_r, gat_w,
           gat_att_src, gat_att_dst, gat_bias):
    src = edge_index[0].astype(jnp.int32)
    dst = edge_index[1].astype(jnp.int32)
    xp = jnp.pad(x, ((0, NP - N), (0, 0)))

    pad1 = EP1 - E
    src1 = jnp.concatenate([src, jnp.zeros((pad1,), jnp.int32)])
    dst1 = jnp.concatenate([dst, jnp.full((pad1,), DUMMY, jnp.int32)])

    loops = jnp.arange(N, dtype=jnp.int32)
    pad2 = EP2 - (E + N)
    src2 = jnp.concatenate([src, loops, jnp.zeros((pad2,), jnp.int32)])
    dst2 = jnp.concatenate([dst, loops, jnp.full((pad2,), DUMMY, jnp.int32)])

    y, r = _sage_mm(xp, sage_w_l.T, sage_w_r.T, sage_b_l)
    accp, cntp = _sage_sc(y, src1, dst1)
    h2, a_s2, a_d2, cs = _mid_tc(accp, cntp, r, gat_w.T,
                                 gat_att_src, gat_att_dst)
    accp2, sp = _gat_sc(h2, a_s2.reshape(NP), a_d2.reshape(NP),
                        cs.reshape(D), src2, dst2)
    return _fin_tc(accp2, sp, gat_bias)
